# baseline (speedup 1.0000x reference)
import functools

import jax
import jax.numpy as jnp
from jax import lax
from jax.experimental import pallas as pl
from jax.experimental.pallas import tpu as pltpu
from jax.experimental.pallas import tpu_sc as plsc

_NC = 2
_NS = 16
_NW = _NC * _NS
_LANES = 128
_GRP = 16
_CHUNK = _LANES * _GRP
_R = 32


def _sc_gather(ids2d, table):
    n_rows = ids2d.shape[0] * ids2d.shape[1]
    rows_per_w = n_rows // _NW
    id_rows_per_w = rows_per_w // _LANES
    n_chunks = rows_per_w // _CHUNK
    mesh = plsc.VectorSubcoreMesh(core_axis_name="c", subcore_axis_name="s")

    @functools.partial(
        pl.kernel,
        mesh=mesh,
        out_type=jax.ShapeDtypeStruct((n_rows, _R), jnp.float32),
        scratch_types=[
            pltpu.VMEM((_GRP, _LANES), jnp.int32),
            pltpu.VMEM((_CHUNK, _R), jnp.float32),
            pltpu.SemaphoreType.DMA,
        ],
        compiler_params=pltpu.CompilerParams(use_tc_tiling_on_sc=False),
    )
    def gather_kernel(ids_hbm, table_hbm, out_hbm, idx_v, rows_v, sem):
        wid = lax.axis_index("s") * _NC + lax.axis_index("c")

        def chunk(j, carry):
            id_row = wid * id_rows_per_w + j * _GRP
            pltpu.sync_copy(ids_hbm.at[pl.ds(id_row, _GRP)], idx_v)
            descs = [
                pltpu.async_copy(
                    table_hbm.at[idx_v.at[g]],
                    rows_v.at[pl.ds(g * _LANES, _LANES)],
                    sem,
                )
                for g in range(_GRP)
            ]
            for d in descs:
                d.wait()
            out_off = wid * rows_per_w + j * _CHUNK
            pltpu.sync_copy(rows_v, out_hbm.at[pl.ds(out_off, _CHUNK)])
            return carry

        lax.fori_loop(0, n_chunks, chunk, 0)

    return gather_kernel(ids2d, table)


_MB = 4096


def _fq_rows(w, q_max):
    amax = jnp.clip(jnp.max(jnp.abs(w), axis=-1, keepdims=True), 1e-8, None)
    scale = amax / q_max
    return jnp.clip(jnp.round(w / scale), -q_max, q_max) * scale


def _tc_quant_matmul(g, b):
    n = g.shape[0]

    def body(g_ref, b_ref, o_ref):
        xq = _fq_rows(g_ref[...], 127.0)
        bq = _fq_rows(b_ref[...], 127.0)
        o_ref[...] = jnp.dot(xq, bq, preferred_element_type=jnp.float32)

    return pl.pallas_call(
        body,
        grid=(n // _MB,),
        in_specs=[
            pl.BlockSpec((_MB, _R), lambda i: (i, 0)),
            pl.BlockSpec((_R, 64), lambda i: (0, 0)),
        ],
        out_specs=pl.BlockSpec((_MB, 64), lambda i: (i, 0)),
        out_shape=jax.ShapeDtypeStruct((n, 64), jnp.float32),
    )(g, b)


def kernel(U, B, local_ids):
    ids2d = local_ids.astype(jnp.int32).reshape(-1, _LANES)
    gathered = _sc_gather(ids2d, U)
    out = _tc_quant_matmul(gathered, B)
    return out.reshape(*local_ids.shape, 64)


# speedup vs baseline: 1.2620x; 1.2620x over previous
import functools

import jax
import jax.numpy as jnp
from jax import lax
from jax.experimental import pallas as pl
from jax.experimental.pallas import tpu as pltpu
from jax.experimental.pallas import tpu_sc as plsc

_NC = 2
_NS = 16
_NW = _NC * _NS
_LANES = 128
_GRP = 16
_CHUNK = _LANES * _GRP
_R = 32
_PACK = _LANES // _R


def _sc_gather(ids2d, table):
    n_rows = ids2d.shape[0] * ids2d.shape[1]
    rows_per_w = n_rows // _NW
    id_rows_per_w = rows_per_w // _LANES
    n_chunks = rows_per_w // _CHUNK
    mesh = plsc.VectorSubcoreMesh(core_axis_name="c", subcore_axis_name="s")

    @functools.partial(
        pl.kernel,
        mesh=mesh,
        out_type=jax.ShapeDtypeStruct((n_rows, _R), jnp.float32),
        scratch_types=[
            pltpu.VMEM((_GRP, _LANES), jnp.int32),
            pltpu.VMEM((_CHUNK, _R), jnp.float32),
            pltpu.SemaphoreType.DMA,
        ],
        compiler_params=pltpu.CompilerParams(use_tc_tiling_on_sc=False),
    )
    def gather_kernel(ids_hbm, table_hbm, out_hbm, idx_v, rows_v, sem):
        wid = lax.axis_index("s") * _NC + lax.axis_index("c")

        def chunk(j, carry):
            id_row = wid * id_rows_per_w + j * _GRP
            pltpu.sync_copy(ids_hbm.at[pl.ds(id_row, _GRP)], idx_v)
            descs = [
                pltpu.async_copy(
                    table_hbm.at[idx_v.at[g]],
                    rows_v.at[pl.ds(g * _LANES, _LANES)],
                    sem,
                )
                for g in range(_GRP)
            ]
            for d in descs:
                d.wait()
            out_off = wid * rows_per_w + j * _CHUNK
            pltpu.sync_copy(rows_v, out_hbm.at[pl.ds(out_off, _CHUNK)])
            return carry

        lax.fori_loop(0, n_chunks, chunk, 0)

    return gather_kernel(ids2d, table)


_MB = 4096


def _fq_rows(w, q_max):
    amax = jnp.clip(jnp.max(jnp.abs(w), axis=-1, keepdims=True), 1e-8, None)
    scale = amax / q_max
    return jnp.clip(jnp.round(w / scale), -q_max, q_max) * scale


def _tc_quant_matmul(g4, b):
    n = g4.shape[0]

    def body(g_ref, b_ref, o_ref):
        x = g_ref[...]
        qs = [_fq_rows(x[:, s * _R:(s + 1) * _R], 127.0) for s in range(_PACK)]
        xq = jnp.concatenate(qs, axis=1)
        bq = _fq_rows(b_ref[...], 127.0)
        zero = jnp.zeros_like(bq)
        bbig = jnp.concatenate(
            [
                jnp.concatenate(
                    [bq if t == s else zero for t in range(_PACK)], axis=1
                )
                for s in range(_PACK)
            ],
            axis=0,
        )
        o_ref[...] = jnp.dot(xq, bbig, preferred_element_type=jnp.float32)

    return pl.pallas_call(
        body,
        grid=(n // _MB,),
        in_specs=[
            pl.BlockSpec((_MB, _LANES), lambda i: (i, 0)),
            pl.BlockSpec((_R, 64), lambda i: (0, 0)),
        ],
        out_specs=pl.BlockSpec((_MB, 2 * _LANES), lambda i: (i, 0)),
        out_shape=jax.ShapeDtypeStruct((n, 2 * _LANES), jnp.float32),
    )(g4, b)


def kernel(U, B, local_ids):
    ids2d = local_ids.astype(jnp.int32).reshape(-1, _LANES)
    gathered4 = _sc_gather(ids2d, U).reshape(-1, _LANES)
    out4 = _tc_quant_matmul(gathered4, B)
    return out4.reshape(*local_ids.shape, 64)


# speedup vs baseline: 2.4987x; 1.9799x over previous
import functools

import jax
import jax.numpy as jnp
from jax import lax
from jax.experimental import pallas as pl
from jax.experimental.pallas import tpu as pltpu
from jax.experimental.pallas import tpu_sc as plsc

_NC = 2
_NS = 16
_NW = _NC * _NS
_LANES = 128
_GRP = 16
_CHUNK = _LANES * _GRP
_R = 32
_PACK = _LANES // _R


def _sc_gather(ids2d, table):
    n_rows = ids2d.shape[0] * ids2d.shape[1]
    rows_per_w = n_rows // _NW
    id_rows_per_w = rows_per_w // _LANES
    n_chunks = rows_per_w // _CHUNK
    mesh = plsc.VectorSubcoreMesh(core_axis_name="c", subcore_axis_name="s")

    @functools.partial(
        pl.kernel,
        mesh=mesh,
        out_type=jax.ShapeDtypeStruct((n_rows, _R), jnp.float32),
        scratch_types=[
            pltpu.VMEM((_GRP, _LANES), jnp.int32),
            pltpu.VMEM((_CHUNK, _R), jnp.float32),
            pltpu.SemaphoreType.DMA,
        ],
        compiler_params=pltpu.CompilerParams(use_tc_tiling_on_sc=False),
    )
    def gather_kernel(ids_hbm, table_hbm, out_hbm, idx_v, rows_v, sem):
        wid = lax.axis_index("s") * _NC + lax.axis_index("c")

        def chunk(j, carry):
            id_row = wid * id_rows_per_w + j * _GRP
            pltpu.sync_copy(ids_hbm.at[pl.ds(id_row, _GRP)], idx_v)
            descs = [
                pltpu.async_copy(
                    table_hbm.at[idx_v.at[g]],
                    rows_v.at[pl.ds(g * _LANES, _LANES)],
                    sem,
                )
                for g in range(_GRP)
            ]
            for d in descs:
                d.wait()
            out_off = wid * rows_per_w + j * _CHUNK
            pltpu.sync_copy(rows_v, out_hbm.at[pl.ds(out_off, _CHUNK)])
            return carry

        lax.fori_loop(0, n_chunks, chunk, 0)

    return gather_kernel(ids2d, table)


_MB = 4096
_QNB = 32768


def _fq_rows(w, q_max):
    amax = jnp.clip(jnp.max(jnp.abs(w), axis=-1, keepdims=True), 1e-8, None)
    scale = amax / q_max
    return jnp.clip(jnp.round(w / scale), -q_max, q_max) * scale


_QL = _QNB // 4


def _tc_quant_pack(ut):
    n = ut.shape[1]
    grid = (n + _QNB - 1) // _QNB

    def body(u_ref, o_ref):
        x = u_ref[...]
        amax = jnp.clip(jnp.max(jnp.abs(x), axis=0, keepdims=True), 1e-8, None)
        scale = amax / 127.0
        q = jnp.clip(jnp.round(x / scale), -127.0, 127.0) * scale
        stacked = jnp.concatenate(
            [q[:, p * _QL:(p + 1) * _QL] for p in range(4)], axis=0
        )
        o_ref[...] = stacked.T

    return pl.pallas_call(
        body,
        grid=(grid,),
        in_specs=[pl.BlockSpec((_R, _QNB), lambda i: (0, i))],
        out_specs=pl.BlockSpec((_QL, _LANES), lambda i: (i, 0)),
        out_shape=jax.ShapeDtypeStruct((grid * _QL, _LANES), jnp.float32),
    )(ut)


def _pack_perm(g):
    b = g // _QNB
    c = g % _QNB
    return b * _QNB + 4 * (c % _QL) + c // _QL


def _tc_matmul(g4, b):
    n = g4.shape[0]

    def body(g_ref, b_ref, o_ref):
        xq = g_ref[...]
        bq = _fq_rows(b_ref[...], 127.0)
        zero = jnp.zeros_like(bq)
        bbig = jnp.concatenate(
            [
                jnp.concatenate(
                    [bq if t == s else zero for t in range(_PACK)], axis=1
                )
                for s in range(_PACK)
            ],
            axis=0,
        )
        o_ref[...] = jnp.dot(xq, bbig, preferred_element_type=jnp.float32)

    return pl.pallas_call(
        body,
        grid=(n // _MB,),
        in_specs=[
            pl.BlockSpec((_MB, _LANES), lambda i: (i, 0)),
            pl.BlockSpec((_R, 64), lambda i: (0, 0)),
        ],
        out_specs=pl.BlockSpec((_MB, 2 * _LANES), lambda i: (i, 0)),
        out_shape=jax.ShapeDtypeStruct((n, 2 * _LANES), jnp.float32),
    )(g4, b)


def kernel(U, B, local_ids):
    ids = _pack_perm(local_ids.astype(jnp.int32))
    ids2d = ids.reshape(-1, _LANES)
    uq4 = _tc_quant_pack(U.T)
    uq = uq4.reshape(uq4.shape[0] * _PACK, _R)
    gathered4 = _sc_gather(ids2d, uq).reshape(-1, _LANES)
    out4 = _tc_matmul(gathered4, B)
    return out4.reshape(*local_ids.shape, 64)


# speedup vs baseline: 3.7840x; 1.5144x over previous
import functools

import jax
import jax.numpy as jnp
from jax import lax
from jax.experimental import pallas as pl
from jax.experimental.pallas import tpu as pltpu
from jax.experimental.pallas import tpu_sc as plsc

_NC = 2
_NS = 16
_NW = _NC * _NS
_LANES = 128
_GRP = 16
_CHUNK = _LANES * _GRP
_R = 32
_PACK = _LANES // _R


def _sc_gather(ids2d, table):
    n_rows = ids2d.shape[0] * ids2d.shape[1]
    rows_per_w = n_rows // _NW
    id_rows_per_w = rows_per_w // _LANES
    n_chunks = rows_per_w // _CHUNK
    mesh = plsc.VectorSubcoreMesh(core_axis_name="c", subcore_axis_name="s")

    @functools.partial(
        pl.kernel,
        mesh=mesh,
        out_type=jax.ShapeDtypeStruct((n_rows, _R), jnp.float32),
        scratch_types=[
            pltpu.VMEM((_GRP, _LANES), jnp.int32),
            pltpu.VMEM((_CHUNK, _R), jnp.float32),
            pltpu.SemaphoreType.DMA,
        ],
        compiler_params=pltpu.CompilerParams(use_tc_tiling_on_sc=False),
    )
    def gather_kernel(ids_hbm, table_hbm, out_hbm, idx_v, rows_v, sem):
        wid = lax.axis_index("s") * _NC + lax.axis_index("c")

        def chunk(j, carry):
            id_row = wid * id_rows_per_w + j * _GRP
            pltpu.sync_copy(ids_hbm.at[pl.ds(id_row, _GRP)], idx_v)
            descs = [
                pltpu.async_copy(
                    table_hbm.at[idx_v.at[g]],
                    rows_v.at[pl.ds(g * _LANES, _LANES)],
                    sem,
                )
                for g in range(_GRP)
            ]
            for d in descs:
                d.wait()
            out_off = wid * rows_per_w + j * _CHUNK
            pltpu.sync_copy(rows_v, out_hbm.at[pl.ds(out_off, _CHUNK)])
            return carry

        lax.fori_loop(0, n_chunks, chunk, 0)

    return gather_kernel(ids2d, table)


_MB = 4096
_QNB = 32768


def _fq_rows(w, q_max):
    amax = jnp.clip(jnp.max(jnp.abs(w), axis=-1, keepdims=True), 1e-8, None)
    scale = amax / q_max
    return jnp.clip(jnp.round(w / scale), -q_max, q_max) * scale


_QL = _QNB // 4


def _tc_quant_pack(ut):
    n = ut.shape[1]
    grid = (n + _QNB - 1) // _QNB

    def body(u_ref, o_ref):
        x = u_ref[...]
        amax = jnp.clip(jnp.max(jnp.abs(x), axis=0, keepdims=True), 1e-8, None)
        scale = amax / 127.0
        q = jnp.clip(jnp.round(x / scale), -127.0, 127.0) * scale
        stacked = jnp.concatenate(
            [q[:, p * _QL:(p + 1) * _QL] for p in range(4)], axis=0
        )
        o_ref[...] = stacked.T

    return pl.pallas_call(
        body,
        grid=(grid,),
        in_specs=[pl.BlockSpec((_R, _QNB), lambda i: (0, i))],
        out_specs=pl.BlockSpec((_QL, _LANES), lambda i: (i, 0)),
        out_shape=jax.ShapeDtypeStruct((grid * _QL, _LANES), jnp.float32),
    )(ut)


def _pack_perm(g):
    b = g // _QNB
    c = g % _QNB
    return b * _QNB + 4 * (c % _QL) + c // _QL


_IB = 4096


def _tc_matmul_t(g4, b, n_tok, n_slot):
    nib = n_tok // _IB

    def body(g_ref, b_ref, o_ref):
        xq4 = g_ref[...]
        xqt = xq4.T
        xt = jnp.concatenate(
            [xqt[_R * p:_R * (p + 1)] for p in range(_PACK)], axis=1
        )
        bq = _fq_rows(b_ref[...], 127.0)
        w = jnp.dot(bq.T, xt, preferred_element_type=jnp.float32)
        o_ref[...] = w.reshape(1, 64, _IB)

    return pl.pallas_call(
        body,
        grid=(n_slot, nib),
        in_specs=[
            pl.BlockSpec((_IB // 4, _LANES), lambda j, ib: (j * nib + ib, 0)),
            pl.BlockSpec((_R, 64), lambda j, ib: (0, 0)),
        ],
        out_specs=pl.BlockSpec((1, 64, _IB), lambda j, ib: (j, 0, ib)),
        out_shape=jax.ShapeDtypeStruct((n_slot, 64, n_tok), jnp.float32),
    )(g4, b)


def kernel(U, B, local_ids):
    n_tok, n_slot = local_ids.shape
    ids_req = (
        local_ids.astype(jnp.int32).T
        .reshape(n_slot, n_tok // _IB, _PACK, _IB // _PACK)
        .transpose(0, 1, 3, 2)
        .reshape(-1)
    )
    ids2d = _pack_perm(ids_req).reshape(-1, _LANES)
    uq4 = _tc_quant_pack(U.T)
    uq = uq4.reshape(uq4.shape[0] * _PACK, _R)
    gathered4 = _sc_gather(ids2d, uq).reshape(-1, _LANES)
    out_t = _tc_matmul_t(gathered4, B, n_tok, n_slot)
    return jnp.transpose(out_t, (2, 0, 1))


# speedup vs baseline: 4.5385x; 1.1994x over previous
import functools

import jax
import jax.numpy as jnp
from jax import lax
from jax.experimental import pallas as pl
from jax.experimental.pallas import tpu as pltpu
from jax.experimental.pallas import tpu_sc as plsc

_NC = 2
_NS = 16
_NW = _NC * _NS
_LANES = 128
_GRP = 16
_CHUNK = _LANES * _GRP
_R = 32
_W = _R // 2
_PACKW = _LANES // _W


def _sc_gather(ids2d, table):
    n_rows = ids2d.shape[0] * ids2d.shape[1]
    rows_per_w = n_rows // _NW
    id_rows_per_w = rows_per_w // _LANES
    n_chunks = rows_per_w // _CHUNK
    mesh = plsc.VectorSubcoreMesh(core_axis_name="c", subcore_axis_name="s")

    @functools.partial(
        pl.kernel,
        mesh=mesh,
        out_type=jax.ShapeDtypeStruct((n_rows, _W), jnp.uint32),
        scratch_types=[
            pltpu.VMEM((_GRP, _LANES), jnp.int32),
            pltpu.VMEM((_CHUNK, _W), jnp.uint32),
            pltpu.SemaphoreType.DMA,
        ],
        compiler_params=pltpu.CompilerParams(use_tc_tiling_on_sc=False),
    )
    def gather_kernel(ids_hbm, table_hbm, out_hbm, idx_v, rows_v, sem):
        wid = lax.axis_index("s") * _NC + lax.axis_index("c")

        def chunk(j, carry):
            id_row = wid * id_rows_per_w + j * _GRP
            pltpu.sync_copy(ids_hbm.at[pl.ds(id_row, _GRP)], idx_v)
            descs = [
                pltpu.async_copy(
                    table_hbm.at[idx_v.at[g]],
                    rows_v.at[pl.ds(g * _LANES, _LANES)],
                    sem,
                )
                for g in range(_GRP)
            ]
            for d in descs:
                d.wait()
            out_off = wid * rows_per_w + j * _CHUNK
            pltpu.sync_copy(rows_v, out_hbm.at[pl.ds(out_off, _CHUNK)])
            return carry

        lax.fori_loop(0, n_chunks, chunk, 0)

    return gather_kernel(ids2d, table)


_QNB = 32768
_QL = _QNB // _PACKW


def _fq_rows(w, q_max):
    amax = jnp.clip(jnp.max(jnp.abs(w), axis=-1, keepdims=True), 1e-8, None)
    scale = amax / q_max
    return jnp.clip(jnp.round(w / scale), -q_max, q_max) * scale


def _tc_quant_pack(ut):
    n = ut.shape[1]
    grid = (n + _QNB - 1) // _QNB

    def body(u_ref, o_ref):
        x = u_ref[...]
        amax = jnp.clip(jnp.max(jnp.abs(x), axis=0, keepdims=True), 1e-8, None)
        scale = amax / 127.0
        q = jnp.clip(jnp.round(x / scale), -127.0, 127.0) * scale
        u = lax.bitcast_convert_type(q, jnp.uint32)
        rne = (u + jnp.uint32(0x7FFF) + ((u >> 16) & jnp.uint32(1))) >> 16
        w = rne[:_W] | (rne[_W:] << 16)
        stacked = jnp.concatenate(
            [w[:, p * _QL:(p + 1) * _QL] for p in range(_PACKW)], axis=0
        )
        o_ref[...] = stacked.T

    return pl.pallas_call(
        body,
        grid=(grid,),
        in_specs=[pl.BlockSpec((_R, _QNB), lambda i: (0, i))],
        out_specs=pl.BlockSpec((_QL, _LANES), lambda i: (i, 0)),
        out_shape=jax.ShapeDtypeStruct((grid * _QL, _LANES), jnp.uint32),
    )(ut)


def _pack_perm(g):
    b = g // _QNB
    c = g % _QNB
    return b * _QNB + _PACKW * (c % _QL) + c // _QL


_IB = 4096


def _tc_matmul_t(g4, b, n_tok, n_slot):
    nib = n_tok // _IB

    def body(g_ref, b_ref, o_ref):
        xw = g_ref[...]
        xwt = xw.T
        w16 = jnp.concatenate(
            [xwt[_W * p:_W * (p + 1)] for p in range(_PACKW)], axis=1
        )
        lo = lax.bitcast_convert_type(w16 << 16, jnp.float32)
        hi = lax.bitcast_convert_type(w16 & jnp.uint32(0xFFFF0000), jnp.float32)
        xt = jnp.concatenate([lo, hi], axis=0)
        bq = _fq_rows(b_ref[...], 127.0)
        w = jnp.dot(bq.T, xt, preferred_element_type=jnp.float32)
        o_ref[...] = w.reshape(1, 64, _IB)

    return pl.pallas_call(
        body,
        grid=(n_slot, nib),
        in_specs=[
            pl.BlockSpec((_IB // _PACKW, _LANES), lambda j, ib: (j * nib + ib, 0)),
            pl.BlockSpec((_R, 64), lambda j, ib: (0, 0)),
        ],
        out_specs=pl.BlockSpec((1, 64, _IB), lambda j, ib: (j, 0, ib)),
        out_shape=jax.ShapeDtypeStruct((n_slot, 64, n_tok), jnp.float32),
    )(g4, b)


def kernel(U, B, local_ids):
    n_tok, n_slot = local_ids.shape
    ids_req = (
        local_ids.astype(jnp.int32).T
        .reshape(n_slot, n_tok // _IB, _PACKW, _IB // _PACKW)
        .transpose(0, 1, 3, 2)
        .reshape(-1)
    )
    ids2d = _pack_perm(ids_req).reshape(-1, _LANES)
    uq4 = _tc_quant_pack(U.T)
    uq = uq4.reshape(uq4.shape[0] * _PACKW, _W)
    gathered4 = _sc_gather(ids2d, uq).reshape(-1, _LANES)
    out_t = _tc_matmul_t(gathered4, B, n_tok, n_slot)
    return jnp.transpose(out_t, (2, 0, 1))


# speedup vs baseline: 5.1938x; 1.1444x over previous
import functools

import jax
import jax.numpy as jnp
from jax import lax
from jax.experimental import pallas as pl
from jax.experimental.pallas import tpu as pltpu
from jax.experimental.pallas import tpu_sc as plsc

_NC = 2
_NS = 16
_NW = _NC * _NS
_LANES = 128
_GRP = 16
_CHUNK = _LANES * _GRP
_R = 32
_W = _R // 2
_PACKW = _LANES // _W


def _sc_gather(ids2d, table):
    n_rows = ids2d.shape[0] * ids2d.shape[1]
    rows_per_w = n_rows // _NW
    id_rows_per_w = rows_per_w // _LANES
    n_chunks = rows_per_w // _CHUNK
    mesh = plsc.VectorSubcoreMesh(core_axis_name="c", subcore_axis_name="s")

    @functools.partial(
        pl.kernel,
        mesh=mesh,
        out_type=jax.ShapeDtypeStruct((n_rows, _W), jnp.uint32),
        scratch_types=[
            pltpu.VMEM((_GRP, _LANES), jnp.int32),
            pltpu.VMEM((_CHUNK, _W), jnp.uint32),
            pltpu.SemaphoreType.DMA,
        ],
        compiler_params=pltpu.CompilerParams(use_tc_tiling_on_sc=False),
    )
    def gather_kernel(ids_hbm, table_hbm, out_hbm, idx_v, rows_v, sem):
        wid = lax.axis_index("s") * _NC + lax.axis_index("c")

        def chunk(j, carry):
            id_row = wid * id_rows_per_w + j * _GRP
            pltpu.sync_copy(ids_hbm.at[pl.ds(id_row, _GRP)], idx_v)
            descs = [
                pltpu.async_copy(
                    table_hbm.at[idx_v.at[g]],
                    rows_v.at[pl.ds(g * _LANES, _LANES)],
                    sem,
                )
                for g in range(_GRP)
            ]
            for d in descs:
                d.wait()
            out_off = wid * rows_per_w + j * _CHUNK
            pltpu.sync_copy(rows_v, out_hbm.at[pl.ds(out_off, _CHUNK)])
            return carry

        lax.fori_loop(0, n_chunks, chunk, 0)

    return gather_kernel(ids2d, table)


_QNB = 65536
_QL = _QNB // _PACKW


def _fq_rows(w, q_max):
    amax = jnp.clip(jnp.max(jnp.abs(w), axis=-1, keepdims=True), 1e-8, None)
    scale = amax / q_max
    return jnp.clip(jnp.round(w / scale), -q_max, q_max) * scale


def _tc_quant_pack(ut):
    n = ut.shape[1]
    grid = (n + _QNB - 1) // _QNB

    def body(u_ref, o_ref):
        x = u_ref[...]
        amax = jnp.clip(jnp.max(jnp.abs(x), axis=0, keepdims=True), 1e-8, None)
        scale = amax / 127.0
        q = jnp.clip(jnp.round(x / scale), -127.0, 127.0) * scale
        u = lax.bitcast_convert_type(q, jnp.uint32)
        rne = (u + jnp.uint32(0x7FFF) + ((u >> 16) & jnp.uint32(1))) >> 16
        w = rne[:_W] | (rne[_W:] << 16)
        stacked = jnp.concatenate(
            [w[:, p * _QL:(p + 1) * _QL] for p in range(_PACKW)], axis=0
        )
        o_ref[...] = stacked.T

    return pl.pallas_call(
        body,
        grid=(grid,),
        in_specs=[pl.BlockSpec((_R, _QNB), lambda i: (0, i))],
        out_specs=pl.BlockSpec((_QL, _LANES), lambda i: (i, 0)),
        out_shape=jax.ShapeDtypeStruct((grid * _QL, _LANES), jnp.uint32),
    )(ut)


def _pack_perm(g):
    b = g // _QNB
    c = g % _QNB
    return b * _QNB + _PACKW * (c % _QL) + c // _QL


_IB = 8192


def _tc_matmul_t(g4, b, n_tok, n_slot):
    nib = n_tok // _IB

    def body(g_ref, b_ref, o_ref):
        xw = g_ref[...]
        xwt = xw.T
        w16 = jnp.concatenate(
            [xwt[_W * p:_W * (p + 1)] for p in range(_PACKW)], axis=1
        )
        lo = lax.bitcast_convert_type(w16 << 16, jnp.float32)
        hi = lax.bitcast_convert_type(w16 & jnp.uint32(0xFFFF0000), jnp.float32)
        xt = jnp.concatenate([lo, hi], axis=0)
        bq = _fq_rows(b_ref[...], 127.0)
        w = jnp.dot(bq.T, xt, preferred_element_type=jnp.float32)
        o_ref[...] = w.reshape(1, 64, _IB)

    return pl.pallas_call(
        body,
        grid=(n_slot, nib),
        in_specs=[
            pl.BlockSpec((_IB // _PACKW, _LANES), lambda j, ib: (j * nib + ib, 0)),
            pl.BlockSpec((_R, 64), lambda j, ib: (0, 0)),
        ],
        out_specs=pl.BlockSpec((1, 64, _IB), lambda j, ib: (j, 0, ib)),
        out_shape=jax.ShapeDtypeStruct((n_slot, 64, n_tok), jnp.float32),
    )(g4, b)


def kernel(U, B, local_ids):
    n_tok, n_slot = local_ids.shape
    ids_req = (
        local_ids.astype(jnp.int32).T
        .reshape(n_slot, n_tok // _IB, _PACKW, _IB // _PACKW)
        .transpose(0, 1, 3, 2)
        .reshape(-1)
    )
    ids2d = _pack_perm(ids_req).reshape(-1, _LANES)
    uq4 = _tc_quant_pack(U.T)
    uq = uq4.reshape(uq4.shape[0] * _PACKW, _W)
    gathered4 = _sc_gather(ids2d, uq).reshape(-1, _LANES)
    out_t = _tc_matmul_t(gathered4, B, n_tok, n_slot)
    return jnp.transpose(out_t, (2, 0, 1))


# speedup vs baseline: 5.6044x; 1.0791x over previous
import functools

import jax
import jax.numpy as jnp
from jax import lax
from jax.experimental import pallas as pl
from jax.experimental.pallas import tpu as pltpu
from jax.experimental.pallas import tpu_sc as plsc

_NC = 2
_NS = 16
_NW = _NC * _NS
_LANES = 128
_GRP = 16
_CHUNK = _LANES * _GRP
_R = 32
_W = _R // 2
_PACKW = _LANES // _W


def _sc_gather(ids2d, table):
    n_rows = ids2d.shape[0] * ids2d.shape[1]
    rows_per_w = n_rows // _NW
    id_rows_per_w = rows_per_w // _LANES
    n_chunks = rows_per_w // _CHUNK
    mesh = plsc.VectorSubcoreMesh(core_axis_name="c", subcore_axis_name="s")

    @functools.partial(
        pl.kernel,
        mesh=mesh,
        out_type=jax.ShapeDtypeStruct((n_rows, _W), jnp.uint32),
        scratch_types=[
            pltpu.VMEM((_GRP, _LANES), jnp.int32),
            pltpu.VMEM((_CHUNK, _W), jnp.uint32),
            pltpu.SemaphoreType.DMA,
        ],
        compiler_params=pltpu.CompilerParams(use_tc_tiling_on_sc=False),
    )
    def gather_kernel(ids_hbm, table_hbm, out_hbm, idx_v, rows_v, sem):
        wid = lax.axis_index("s") * _NC + lax.axis_index("c")

        def chunk(j, carry):
            id_row = wid * id_rows_per_w + j * _GRP
            pltpu.sync_copy(ids_hbm.at[pl.ds(id_row, _GRP)], idx_v)
            descs = [
                pltpu.async_copy(
                    table_hbm.at[idx_v.at[g]],
                    rows_v.at[pl.ds(g * _LANES, _LANES)],
                    sem,
                )
                for g in range(_GRP)
            ]
            for d in descs:
                d.wait()
            out_off = wid * rows_per_w + j * _CHUNK
            pltpu.sync_copy(rows_v, out_hbm.at[pl.ds(out_off, _CHUNK)])
            return carry

        lax.fori_loop(0, n_chunks, chunk, 0)

    return gather_kernel(ids2d, table)


_QNB = 131072
_QL = _QNB // _PACKW


def _fq_rows(w, q_max):
    amax = jnp.clip(jnp.max(jnp.abs(w), axis=-1, keepdims=True), 1e-8, None)
    scale = amax / q_max
    return jnp.clip(jnp.round(w / scale), -q_max, q_max) * scale


def _tc_quant_pack(ut):
    n = ut.shape[1]
    grid = (n + _QNB - 1) // _QNB

    def body(u_ref, o_ref):
        x = u_ref[...]
        amax = jnp.clip(jnp.max(jnp.abs(x), axis=0, keepdims=True), 1e-8, None)
        scale = amax / 127.0
        q = jnp.clip(jnp.round(x / scale), -127.0, 127.0) * scale
        u = lax.bitcast_convert_type(q, jnp.uint32)
        rne = (u + jnp.uint32(0x7FFF) + ((u >> 16) & jnp.uint32(1))) >> 16
        w = rne[:_W] | (rne[_W:] << 16)
        stacked = jnp.concatenate(
            [w[:, p * _QL:(p + 1) * _QL] for p in range(_PACKW)], axis=0
        )
        o_ref[...] = stacked.T

    return pl.pallas_call(
        body,
        grid=(grid,),
        in_specs=[pl.BlockSpec((_R, _QNB), lambda i: (0, i))],
        out_specs=pl.BlockSpec((_QL, _LANES), lambda i: (i, 0)),
        out_shape=jax.ShapeDtypeStruct((grid * _QL, _LANES), jnp.uint32),
    )(ut)


def _pack_perm(g):
    b = g // _QNB
    c = g % _QNB
    return b * _QNB + _PACKW * (c % _QL) + c // _QL


_IB = 16384


def _tc_matmul_t(g4, b, n_tok, n_slot):
    nib = n_tok // _IB

    def body(g_ref, b_ref, o_ref):
        xw = g_ref[...]
        xwt = xw.T
        w16 = jnp.concatenate(
            [xwt[_W * p:_W * (p + 1)] for p in range(_PACKW)], axis=1
        )
        lo = lax.bitcast_convert_type(w16 << 16, jnp.float32)
        hi = lax.bitcast_convert_type(w16 & jnp.uint32(0xFFFF0000), jnp.float32)
        xt = jnp.concatenate([lo, hi], axis=0)
        bq = _fq_rows(b_ref[...], 127.0)
        w = jnp.dot(bq.T, xt, preferred_element_type=jnp.float32)
        o_ref[...] = w.reshape(1, 64, _IB)

    return pl.pallas_call(
        body,
        grid=(n_slot, nib),
        in_specs=[
            pl.BlockSpec((_IB // _PACKW, _LANES), lambda j, ib: (j * nib + ib, 0)),
            pl.BlockSpec((_R, 64), lambda j, ib: (0, 0)),
        ],
        out_specs=pl.BlockSpec((1, 64, _IB), lambda j, ib: (j, 0, ib)),
        out_shape=jax.ShapeDtypeStruct((n_slot, 64, n_tok), jnp.float32),
    )(g4, b)


def kernel(U, B, local_ids):
    n_tok, n_slot = local_ids.shape
    ids_req = (
        local_ids.astype(jnp.int32).T
        .reshape(n_slot, n_tok // _IB, _PACKW, _IB // _PACKW)
        .transpose(0, 1, 3, 2)
        .reshape(-1)
    )
    ids2d = _pack_perm(ids_req).reshape(-1, _LANES)
    uq4 = _tc_quant_pack(U.T)
    uq = uq4.reshape(uq4.shape[0] * _PACKW, _W)
    gathered4 = _sc_gather(ids2d, uq).reshape(-1, _LANES)
    out_t = _tc_matmul_t(gathered4, B, n_tok, n_slot)
    return jnp.transpose(out_t, (2, 0, 1))
